# baseline (speedup 1.0000x reference)
import jax
import jax.numpy as jnp
from jax import lax
from jax.experimental import pallas as pl
from jax.experimental.pallas import tpu as pltpu

B, M = 256, 20
N = B * M
H, F, G, L = 128, 128, 50, 3
CUTOFF = 5.0
LN2 = 0.6931471805599453
NZ = 100

BM = 16
GRID = B // BM
R = BM * M
P = M * R


def _ssp(x):
    return jnp.maximum(x, 0.0) + jnp.log1p(jnp.exp(-jnp.abs(x))) - LN2


def _block_kernel(z_ref, posr_ref, posm_ref, s_ref, tcat_ref, st_ref, eye_ref,
                  emb_ref,
                  mw1_ref, mb1_ref, mw2_ref, mb2_ref,
                  cw1_ref, cw2_ref, cb2_ref, iw_ref, ib_ref,
                  ow1_ref, ob1_ref, ow2_ref, ob2_ref,
                  out_ref):
    S = s_ref[...]
    Tcat = tcat_ref[...]
    St = st_ref[...]
    eyef = eye_ref[...]

    z = z_ref[...]
    ids = lax.broadcasted_iota(jnp.int32, (R, NZ), 1)
    oh = (ids == z).astype(jnp.float32)
    h = jnp.dot(oh, emb_ref[...], preferred_element_type=jnp.float32)

    d2 = jnp.zeros((R, M), jnp.float32)
    for c in range(3):
        pc_self = posr_ref[c]
        pc_mol = posm_ref[c]
        pc_partner = jnp.dot(S, pc_mol, preferred_element_type=jnp.float32)
        dc = pc_self - pc_partner
        d2 = d2 + dc * dc
    dist = jnp.sqrt(d2)
    cosw = 0.5 * (jnp.cos(dist * (jnp.pi / CUTOFF)) + 1.0)
    wc = jnp.where((dist < CUTOFF) & (eyef == 0.0), cosw, 0.0)

    delta = CUTOFF / (G - 1)
    offs = lax.broadcasted_iota(jnp.int32, (1, G), 1).astype(jnp.float32) * delta
    coeff = -0.5 / (delta * delta)
    pieces = []
    for j in range(M):
        dj = dist[:, j:j + 1]
        pieces.append(jnp.exp(coeff * (dj - offs) ** 2))
    rbf = jnp.concatenate(pieces, axis=0)

    hh = h
    rbf16 = rbf.astype(jnp.bfloat16)
    for l in range(L):
        t = jnp.dot(rbf16, mw1_ref[l].astype(jnp.bfloat16),
                    preferred_element_type=jnp.float32)
        t = _ssp(t + mb1_ref[l:l + 1, :])
        t = jnp.dot(t.astype(jnp.bfloat16), mw2_ref[l].astype(jnp.bfloat16),
                    preferred_element_type=jnp.float32)
        t = t + mb2_ref[l:l + 1, :]
        xj = jnp.dot(hh, cw1_ref[l], preferred_element_type=jnp.float32)
        xjsel = jnp.dot(Tcat, xj, preferred_element_type=jnp.float32)
        msg = jnp.zeros((R, F), jnp.float32)
        for j in range(M):
            xjrow = jnp.dot(S, xjsel[j * BM:(j + 1) * BM, :],
                            preferred_element_type=jnp.float32)
            msg = msg + t[j * R:(j + 1) * R, :] * (wc[:, j:j + 1] * xjrow)
        msg = jnp.dot(msg, cw2_ref[l], preferred_element_type=jnp.float32)
        msg = _ssp(msg + cb2_ref[l:l + 1, :])
        msg = jnp.dot(msg, iw_ref[l], preferred_element_type=jnp.float32)
        hh = hh + msg + ib_ref[l:l + 1, :]

    o = _ssp(jnp.dot(hh, ow1_ref[...], preferred_element_type=jnp.float32)
             + ob1_ref[...])
    o = jnp.dot(o, ow2_ref[...], preferred_element_type=jnp.float32) + ob2_ref[...]
    out_ref[...] = jnp.dot(St, o, preferred_element_type=jnp.float32)


@jax.jit
def kernel(z, pos, batch, emb, mlp_w1, mlp_b1, mlp_w2, mlp_b2,
           conv_w1, conv_w2, conv_b2, int_w, int_b,
           out_w1, out_b1, out_w2, out_b2):
    del batch
    z2 = z.reshape(N, 1)
    posr = pos.T.reshape(3, N, 1)
    posm = pos.reshape(B, M, 3).transpose(2, 0, 1)
    ob1 = out_b1.reshape(1, H // 2)
    ob2 = out_b2.reshape(1, 1)

    r_i = jnp.arange(R, dtype=jnp.int32)
    mol = r_i // M
    atom = r_i % M
    S = (mol[:, None] == jnp.arange(BM, dtype=jnp.int32)[None, :]).astype(jnp.float32)
    q_j = jnp.arange(R, dtype=jnp.int32) // BM
    q_bm = jnp.arange(R, dtype=jnp.int32) % BM
    Tcat = ((mol[None, :] == q_bm[:, None])
            & (atom[None, :] == q_j[:, None])).astype(jnp.float32)
    St = S.T
    eyef = (atom[:, None] == jnp.arange(M, dtype=jnp.int32)[None, :]).astype(jnp.float32)

    full = lambda a: pl.BlockSpec(a.shape, lambda i: (0,) * a.ndim)
    out = pl.pallas_call(
        _block_kernel,
        grid=(GRID,),
        in_specs=[
            pl.BlockSpec((R, 1), lambda i: (i, 0)),
            pl.BlockSpec((3, R, 1), lambda i: (0, i, 0)),
            pl.BlockSpec((3, BM, M), lambda i: (0, i, 0)),
            full(S), full(Tcat), full(St), full(eyef),
            full(emb),
            full(mlp_w1), full(mlp_b1), full(mlp_w2), full(mlp_b2),
            full(conv_w1), full(conv_w2), full(conv_b2),
            full(int_w), full(int_b),
            full(out_w1), full(ob1), full(out_w2), full(ob2),
        ],
        out_specs=pl.BlockSpec((BM, 1), lambda i: (i, 0)),
        out_shape=jax.ShapeDtypeStruct((B, 1), jnp.float32),
        compiler_params=pltpu.CompilerParams(
            dimension_semantics=("parallel",)),
    )(z2, posr, posm, S, Tcat, St, eyef,
      emb, mlp_w1, mlp_b1, mlp_w2, mlp_b2,
      conv_w1, conv_w2, conv_b2, int_w, int_b, out_w1, ob1, out_w2, ob2)
    return out


# speedup vs baseline: 1.2192x; 1.2192x over previous
import jax
import jax.numpy as jnp
from jax import lax
from jax.experimental import pallas as pl
from jax.experimental.pallas import tpu as pltpu

B, M = 256, 20
N = B * M
H, F, G, L = 128, 128, 50, 3
CUTOFF = 5.0
LN2 = 0.6931471805599453
NZ = 100

BM = 16
GRID = B // BM
R = BM * M
P = M * R


def _ssp(x):
    return jnp.maximum(x, 0.0) + jnp.log1p(jnp.exp(-jnp.abs(x))) - LN2


def _block_kernel(z_ref, posr_ref, posm_ref, s_ref, tcat_ref, st_ref, eye_ref,
                  emb_ref,
                  mw1_ref, mb1_ref, mw2_ref, mb2_ref,
                  cw1_ref, cw2_ref, cb2_ref, iw_ref, ib_ref,
                  ow1_ref, ob1_ref, ow2_ref, ob2_ref,
                  out_ref):
    S = s_ref[...]
    Tcat = tcat_ref[...]
    St = st_ref[...]
    eyef = eye_ref[...]

    z = z_ref[...]
    ids = lax.broadcasted_iota(jnp.int32, (R, NZ), 1)
    oh = (ids == z).astype(jnp.float32)
    h = jnp.dot(oh, emb_ref[...], preferred_element_type=jnp.float32)

    d2 = jnp.zeros((R, M), jnp.float32)
    for c in range(3):
        pc_self = posr_ref[c]
        pc_mol = posm_ref[c]
        pc_partner = jnp.dot(S, pc_mol, preferred_element_type=jnp.float32)
        dc = pc_self - pc_partner
        d2 = d2 + dc * dc
    dist = jnp.sqrt(d2)
    cosw = 0.5 * (jnp.cos(dist * (jnp.pi / CUTOFF)) + 1.0)
    wc = jnp.where((dist < CUTOFF) & (eyef == 0.0), cosw, 0.0)

    delta = CUTOFF / (G - 1)
    offs = lax.broadcasted_iota(jnp.int32, (1, G), 1).astype(jnp.float32) * delta
    coeff = -0.5 / (delta * delta)
    pieces = []
    for j in range(M):
        dj = dist[:, j:j + 1]
        pieces.append(jnp.exp(coeff * (dj - offs) ** 2))
    rbf = jnp.concatenate(pieces, axis=0)

    rbf16 = rbf.astype(jnp.bfloat16)
    tf = []
    for l in range(L):
        u = jnp.dot(rbf16, mw1_ref[l].astype(jnp.bfloat16),
                    preferred_element_type=jnp.float32)
        x = (u + mb1_ref[l:l + 1, :]).astype(jnp.bfloat16)
        s = (jnp.maximum(x, jnp.bfloat16(0.0)) - jnp.bfloat16(LN2)
             + jnp.log1p(jnp.exp(-jnp.abs(x))))
        t = jnp.dot(s, mw2_ref[l].astype(jnp.bfloat16),
                    preferred_element_type=jnp.float32)
        tf.append(t + mb2_ref[l:l + 1, :])

    S16 = S.astype(jnp.bfloat16)
    Tcat16 = Tcat.astype(jnp.bfloat16)
    hh = h
    for l in range(L):
        xj = jnp.dot(hh.astype(jnp.bfloat16), cw1_ref[l].astype(jnp.bfloat16),
                     preferred_element_type=jnp.float32)
        xjsel = jnp.dot(Tcat16, xj.astype(jnp.bfloat16),
                        preferred_element_type=jnp.float32)
        xjsel16 = xjsel.astype(jnp.bfloat16)
        t = tf[l]
        msg = jnp.zeros((R, F), jnp.float32)
        for j in range(M):
            xjrow = jnp.dot(S16, xjsel16[j * BM:(j + 1) * BM, :],
                            preferred_element_type=jnp.float32)
            msg = msg + t[j * R:(j + 1) * R, :] * (wc[:, j:j + 1] * xjrow)
        msg = jnp.dot(msg.astype(jnp.bfloat16), cw2_ref[l].astype(jnp.bfloat16),
                      preferred_element_type=jnp.float32)
        msg = _ssp(msg + cb2_ref[l:l + 1, :])
        msg = jnp.dot(msg.astype(jnp.bfloat16), iw_ref[l].astype(jnp.bfloat16),
                      preferred_element_type=jnp.float32)
        hh = hh + msg + ib_ref[l:l + 1, :]

    o = _ssp(jnp.dot(hh, ow1_ref[...], preferred_element_type=jnp.float32)
             + ob1_ref[...])
    o = jnp.dot(o, ow2_ref[...], preferred_element_type=jnp.float32) + ob2_ref[...]
    out_ref[...] = jnp.dot(St, o, preferred_element_type=jnp.float32)


@jax.jit
def kernel(z, pos, batch, emb, mlp_w1, mlp_b1, mlp_w2, mlp_b2,
           conv_w1, conv_w2, conv_b2, int_w, int_b,
           out_w1, out_b1, out_w2, out_b2):
    del batch
    z2 = z.reshape(N, 1)
    posr = pos.T.reshape(3, N, 1)
    posm = pos.reshape(B, M, 3).transpose(2, 0, 1)
    ob1 = out_b1.reshape(1, H // 2)
    ob2 = out_b2.reshape(1, 1)

    r_i = jnp.arange(R, dtype=jnp.int32)
    mol = r_i // M
    atom = r_i % M
    S = (mol[:, None] == jnp.arange(BM, dtype=jnp.int32)[None, :]).astype(jnp.float32)
    q_j = jnp.arange(R, dtype=jnp.int32) // BM
    q_bm = jnp.arange(R, dtype=jnp.int32) % BM
    Tcat = ((mol[None, :] == q_bm[:, None])
            & (atom[None, :] == q_j[:, None])).astype(jnp.float32)
    St = S.T
    eyef = (atom[:, None] == jnp.arange(M, dtype=jnp.int32)[None, :]).astype(jnp.float32)

    full = lambda a: pl.BlockSpec(a.shape, lambda i: (0,) * a.ndim)
    out = pl.pallas_call(
        _block_kernel,
        grid=(GRID,),
        in_specs=[
            pl.BlockSpec((R, 1), lambda i: (i, 0)),
            pl.BlockSpec((3, R, 1), lambda i: (0, i, 0)),
            pl.BlockSpec((3, BM, M), lambda i: (0, i, 0)),
            full(S), full(Tcat), full(St), full(eyef),
            full(emb),
            full(mlp_w1), full(mlp_b1), full(mlp_w2), full(mlp_b2),
            full(conv_w1), full(conv_w2), full(conv_b2),
            full(int_w), full(int_b),
            full(out_w1), full(ob1), full(out_w2), full(ob2),
        ],
        out_specs=pl.BlockSpec((BM, 1), lambda i: (i, 0)),
        out_shape=jax.ShapeDtypeStruct((B, 1), jnp.float32),
        compiler_params=pltpu.CompilerParams(
            dimension_semantics=("parallel",)),
    )(z2, posr, posm, S, Tcat, St, eyef,
      emb, mlp_w1, mlp_b1, mlp_w2, mlp_b2,
      conv_w1, conv_w2, conv_b2, int_w, int_b, out_w1, ob1, out_w2, ob2)
    return out


# speedup vs baseline: 1.2414x; 1.0182x over previous
import jax
import jax.numpy as jnp
from jax import lax
from jax.experimental import pallas as pl
from jax.experimental.pallas import tpu as pltpu

B, M = 256, 20
N = B * M
H, F, G, L = 128, 128, 50, 3
CUTOFF = 5.0
LN2 = 0.6931471805599453
NZ = 100

BM = 16
GRID = B // BM
R = BM * M
P = M * R


def _ssp(x):
    return jnp.maximum(x, 0.0) + jnp.log1p(jnp.exp(-jnp.abs(x))) - LN2


def _block_kernel(z_ref, posr_ref, posm_ref, s_ref, tcat_ref, st_ref, eye_ref,
                  emb_ref,
                  mw1_ref, mb1_ref, mw2_ref, mb2_ref,
                  cw1_ref, cw2_ref, cb2_ref, iw_ref, ib_ref,
                  ow1_ref, ob1_ref, ow2_ref, ob2_ref,
                  out_ref):
    S = s_ref[...]
    Tcat = tcat_ref[...]
    St = st_ref[...]
    eyef = eye_ref[...]

    z = z_ref[...]
    ids = lax.broadcasted_iota(jnp.int32, (R, NZ), 1)
    oh = (ids == z).astype(jnp.float32)
    h = jnp.dot(oh, emb_ref[...], preferred_element_type=jnp.float32)

    d2 = jnp.zeros((R, M), jnp.float32)
    for c in range(3):
        pc_self = posr_ref[c]
        pc_mol = posm_ref[c]
        pc_partner = jnp.dot(S, pc_mol, preferred_element_type=jnp.float32)
        dc = pc_self - pc_partner
        d2 = d2 + dc * dc
    dist = jnp.sqrt(d2)
    cosw = 0.5 * (jnp.cos(dist * (jnp.pi / CUTOFF)) + 1.0)
    wc = jnp.where((dist < CUTOFF) & (eyef == 0.0),
                   cosw, 0.0).astype(jnp.bfloat16)

    delta = CUTOFF / (G - 1)
    offs = lax.broadcasted_iota(jnp.int32, (1, G), 1).astype(jnp.float32) * delta
    coeff = -0.5 / (delta * delta)
    pieces = []
    for j in range(M):
        dj = dist[:, j:j + 1]
        arg = coeff * (dj - offs) ** 2
        pieces.append(jnp.exp(arg.astype(jnp.bfloat16)))
    rbf16 = jnp.concatenate(pieces, axis=0)

    tf = []
    for l in range(L):
        u = jnp.dot(rbf16, mw1_ref[l].astype(jnp.bfloat16),
                    preferred_element_type=jnp.float32)
        x = (u + mb1_ref[l:l + 1, :]).astype(jnp.bfloat16)
        s = (jnp.maximum(x, jnp.bfloat16(0.0)) - jnp.bfloat16(LN2)
             + jnp.log1p(jnp.exp(-jnp.abs(x))))
        t = jnp.dot(s, mw2_ref[l].astype(jnp.bfloat16),
                    preferred_element_type=jnp.float32)
        tf.append((t + mb2_ref[l:l + 1, :]).astype(jnp.bfloat16))

    S16 = S.astype(jnp.bfloat16)
    Tcat16 = Tcat.astype(jnp.bfloat16)
    hh = h
    for l in range(L):
        xj = jnp.dot(hh.astype(jnp.bfloat16), cw1_ref[l].astype(jnp.bfloat16),
                     preferred_element_type=jnp.float32)
        xjsel16 = jnp.dot(Tcat16, xj.astype(jnp.bfloat16),
                          preferred_element_type=jnp.float32
                          ).astype(jnp.bfloat16)
        t = tf[l]
        msg16 = jnp.zeros((R, F), jnp.bfloat16)
        for j in range(M):
            xjrow = jnp.dot(S16, xjsel16[j * BM:(j + 1) * BM, :],
                            preferred_element_type=jnp.float32
                            ).astype(jnp.bfloat16)
            msg16 = msg16 + t[j * R:(j + 1) * R, :] * (wc[:, j:j + 1] * xjrow)
        msg = jnp.dot(msg16, cw2_ref[l].astype(jnp.bfloat16),
                      preferred_element_type=jnp.float32)
        msg = _ssp(msg + cb2_ref[l:l + 1, :])
        msg = jnp.dot(msg.astype(jnp.bfloat16), iw_ref[l].astype(jnp.bfloat16),
                      preferred_element_type=jnp.float32)
        hh = hh + msg + ib_ref[l:l + 1, :]

    o = _ssp(jnp.dot(hh, ow1_ref[...], preferred_element_type=jnp.float32)
             + ob1_ref[...])
    o = jnp.dot(o, ow2_ref[...], preferred_element_type=jnp.float32) + ob2_ref[...]
    out_ref[...] = jnp.dot(St, o, preferred_element_type=jnp.float32)


@jax.jit
def kernel(z, pos, batch, emb, mlp_w1, mlp_b1, mlp_w2, mlp_b2,
           conv_w1, conv_w2, conv_b2, int_w, int_b,
           out_w1, out_b1, out_w2, out_b2):
    del batch
    z2 = z.reshape(N, 1)
    posr = pos.T.reshape(3, N, 1)
    posm = pos.reshape(B, M, 3).transpose(2, 0, 1)
    ob1 = out_b1.reshape(1, H // 2)
    ob2 = out_b2.reshape(1, 1)

    r_i = jnp.arange(R, dtype=jnp.int32)
    mol = r_i // M
    atom = r_i % M
    S = (mol[:, None] == jnp.arange(BM, dtype=jnp.int32)[None, :]).astype(jnp.float32)
    q_j = jnp.arange(R, dtype=jnp.int32) // BM
    q_bm = jnp.arange(R, dtype=jnp.int32) % BM
    Tcat = ((mol[None, :] == q_bm[:, None])
            & (atom[None, :] == q_j[:, None])).astype(jnp.float32)
    St = S.T
    eyef = (atom[:, None] == jnp.arange(M, dtype=jnp.int32)[None, :]).astype(jnp.float32)

    full = lambda a: pl.BlockSpec(a.shape, lambda i: (0,) * a.ndim)
    out = pl.pallas_call(
        _block_kernel,
        grid=(GRID,),
        in_specs=[
            pl.BlockSpec((R, 1), lambda i: (i, 0)),
            pl.BlockSpec((3, R, 1), lambda i: (0, i, 0)),
            pl.BlockSpec((3, BM, M), lambda i: (0, i, 0)),
            full(S), full(Tcat), full(St), full(eyef),
            full(emb),
            full(mlp_w1), full(mlp_b1), full(mlp_w2), full(mlp_b2),
            full(conv_w1), full(conv_w2), full(conv_b2),
            full(int_w), full(int_b),
            full(out_w1), full(ob1), full(out_w2), full(ob2),
        ],
        out_specs=pl.BlockSpec((BM, 1), lambda i: (i, 0)),
        out_shape=jax.ShapeDtypeStruct((B, 1), jnp.float32),
        compiler_params=pltpu.CompilerParams(
            dimension_semantics=("parallel",)),
    )(z2, posr, posm, S, Tcat, St, eyef,
      emb, mlp_w1, mlp_b1, mlp_w2, mlp_b2,
      conv_w1, conv_w2, conv_b2, int_w, int_b, out_w1, ob1, out_w2, ob2)
    return out


# speedup vs baseline: 1.3588x; 1.0946x over previous
import jax
import jax.numpy as jnp
from jax import lax
from jax.experimental import pallas as pl
from jax.experimental.pallas import tpu as pltpu

B, M = 256, 20
N = B * M
H, F, G, L = 128, 128, 50, 3
CUTOFF = 5.0
LN2 = 0.6931471805599453
NZ = 100

BM = 32
GRID = B // BM
R = BM * M
P = M * R


def _ssp(x):
    return jnp.maximum(x, 0.0) + jnp.log1p(jnp.exp(-jnp.abs(x))) - LN2


def _block_kernel(z_ref, posr_ref, posm_ref, s_ref, tcat_ref, st_ref, eye_ref,
                  emb_ref,
                  mw1_ref, mb1_ref, mw2_ref, mb2_ref,
                  cw1_ref, cw2_ref, cb2_ref, iw_ref, ib_ref,
                  ow1_ref, ob1_ref, ow2_ref, ob2_ref,
                  out_ref):
    S = s_ref[...]
    Tcat = tcat_ref[...]
    St = st_ref[...]
    eyef = eye_ref[...]

    z = z_ref[...]
    ids = lax.broadcasted_iota(jnp.int32, (R, NZ), 1)
    oh = (ids == z).astype(jnp.float32)
    h = jnp.dot(oh, emb_ref[...], preferred_element_type=jnp.float32)

    d2 = jnp.zeros((R, M), jnp.float32)
    for c in range(3):
        pc_self = posr_ref[c]
        pc_mol = posm_ref[c]
        pc_partner = jnp.dot(S, pc_mol, preferred_element_type=jnp.float32)
        dc = pc_self - pc_partner
        d2 = d2 + dc * dc
    dist = jnp.sqrt(d2)
    cosw = 0.5 * (jnp.cos(dist * (jnp.pi / CUTOFF)) + 1.0)
    wc = jnp.where((dist < CUTOFF) & (eyef == 0.0),
                   cosw, 0.0).astype(jnp.bfloat16)

    delta = CUTOFF / (G - 1)
    offs = lax.broadcasted_iota(jnp.int32, (1, G), 1).astype(jnp.float32) * delta
    coeff = -0.5 / (delta * delta)
    pieces = []
    for j in range(M):
        dj = dist[:, j:j + 1]
        arg = coeff * (dj - offs) ** 2
        pieces.append(jnp.exp(arg.astype(jnp.bfloat16)))
    rbf16 = jnp.concatenate(pieces, axis=0)

    tf = []
    for l in range(L):
        u = jnp.dot(rbf16, mw1_ref[l].astype(jnp.bfloat16),
                    preferred_element_type=jnp.float32)
        x = (u + mb1_ref[l:l + 1, :]).astype(jnp.bfloat16)
        s = (jnp.maximum(x, jnp.bfloat16(0.0)) - jnp.bfloat16(LN2)
             + jnp.log1p(jnp.exp(-jnp.abs(x))))
        t = jnp.dot(s, mw2_ref[l].astype(jnp.bfloat16),
                    preferred_element_type=jnp.float32)
        tf.append((t + mb2_ref[l:l + 1, :]).astype(jnp.bfloat16))

    S16 = S.astype(jnp.bfloat16)
    Tcat16 = Tcat.astype(jnp.bfloat16)
    hh = h
    for l in range(L):
        xj = jnp.dot(hh.astype(jnp.bfloat16), cw1_ref[l].astype(jnp.bfloat16),
                     preferred_element_type=jnp.float32)
        xjsel16 = jnp.dot(Tcat16, xj.astype(jnp.bfloat16),
                          preferred_element_type=jnp.float32
                          ).astype(jnp.bfloat16)
        t = tf[l]
        msg16 = jnp.zeros((R, F), jnp.bfloat16)
        for j in range(M):
            xjrow = jnp.dot(S16, xjsel16[j * BM:(j + 1) * BM, :],
                            preferred_element_type=jnp.float32
                            ).astype(jnp.bfloat16)
            msg16 = msg16 + t[j * R:(j + 1) * R, :] * (wc[:, j:j + 1] * xjrow)
        msg = jnp.dot(msg16, cw2_ref[l].astype(jnp.bfloat16),
                      preferred_element_type=jnp.float32)
        msg = _ssp(msg + cb2_ref[l:l + 1, :])
        msg = jnp.dot(msg.astype(jnp.bfloat16), iw_ref[l].astype(jnp.bfloat16),
                      preferred_element_type=jnp.float32)
        hh = hh + msg + ib_ref[l:l + 1, :]

    o = _ssp(jnp.dot(hh, ow1_ref[...], preferred_element_type=jnp.float32)
             + ob1_ref[...])
    o = jnp.dot(o, ow2_ref[...], preferred_element_type=jnp.float32) + ob2_ref[...]
    out_ref[...] = jnp.dot(St, o, preferred_element_type=jnp.float32)


@jax.jit
def kernel(z, pos, batch, emb, mlp_w1, mlp_b1, mlp_w2, mlp_b2,
           conv_w1, conv_w2, conv_b2, int_w, int_b,
           out_w1, out_b1, out_w2, out_b2):
    del batch
    z2 = z.reshape(N, 1)
    posr = pos.T.reshape(3, N, 1)
    posm = pos.reshape(B, M, 3).transpose(2, 0, 1)
    ob1 = out_b1.reshape(1, H // 2)
    ob2 = out_b2.reshape(1, 1)

    r_i = jnp.arange(R, dtype=jnp.int32)
    mol = r_i // M
    atom = r_i % M
    S = (mol[:, None] == jnp.arange(BM, dtype=jnp.int32)[None, :]).astype(jnp.float32)
    q_j = jnp.arange(R, dtype=jnp.int32) // BM
    q_bm = jnp.arange(R, dtype=jnp.int32) % BM
    Tcat = ((mol[None, :] == q_bm[:, None])
            & (atom[None, :] == q_j[:, None])).astype(jnp.float32)
    St = S.T
    eyef = (atom[:, None] == jnp.arange(M, dtype=jnp.int32)[None, :]).astype(jnp.float32)

    full = lambda a: pl.BlockSpec(a.shape, lambda i: (0,) * a.ndim)
    out = pl.pallas_call(
        _block_kernel,
        grid=(GRID,),
        in_specs=[
            pl.BlockSpec((R, 1), lambda i: (i, 0)),
            pl.BlockSpec((3, R, 1), lambda i: (0, i, 0)),
            pl.BlockSpec((3, BM, M), lambda i: (0, i, 0)),
            full(S), full(Tcat), full(St), full(eyef),
            full(emb),
            full(mlp_w1), full(mlp_b1), full(mlp_w2), full(mlp_b2),
            full(conv_w1), full(conv_w2), full(conv_b2),
            full(int_w), full(int_b),
            full(out_w1), full(ob1), full(out_w2), full(ob2),
        ],
        out_specs=pl.BlockSpec((BM, 1), lambda i: (i, 0)),
        out_shape=jax.ShapeDtypeStruct((B, 1), jnp.float32),
        compiler_params=pltpu.CompilerParams(
            dimension_semantics=("parallel",)),
    )(z2, posr, posm, S, Tcat, St, eyef,
      emb, mlp_w1, mlp_b1, mlp_w2, mlp_b2,
      conv_w1, conv_w2, conv_b2, int_w, int_b, out_w1, ob1, out_w2, ob2)
    return out


# speedup vs baseline: 1.4488x; 1.0663x over previous
import jax
import jax.numpy as jnp
from jax import lax
from jax.experimental import pallas as pl
from jax.experimental.pallas import tpu as pltpu

B, M = 256, 20
N = B * M
H, F, G, L = 128, 128, 50, 3
CUTOFF = 5.0
LN2 = 0.6931471805599453
NZ = 100

BM = 32
GRID = B // BM
R = BM * M
P = M * R


def _ssp(x):
    return jnp.maximum(x, 0.0) + jnp.log1p(jnp.exp(-jnp.abs(x))) - LN2


def _block_kernel(z_ref, posr_ref, posm_ref, s_ref, tcat_ref, st_ref, eye_ref,
                  emb_ref,
                  mw1_ref, mb1_ref, mw2_ref, mb2_ref,
                  cw1_ref, cw2_ref, cb2_ref, iw_ref, ib_ref,
                  ow1_ref, ob1_ref, ow2_ref, ob2_ref,
                  out_ref):
    S = s_ref[...]
    Tcat = tcat_ref[...]
    St = st_ref[...]
    eyef = eye_ref[...]

    z = z_ref[...]
    ids = lax.broadcasted_iota(jnp.int32, (R, NZ), 1)
    oh = (ids == z).astype(jnp.float32)
    h = jnp.dot(oh, emb_ref[...], preferred_element_type=jnp.float32)

    d2 = jnp.zeros((R, M), jnp.float32)
    for c in range(3):
        pc_self = posr_ref[c]
        pc_mol = posm_ref[c]
        pc_partner = jnp.dot(S, pc_mol, preferred_element_type=jnp.float32)
        dc = pc_self - pc_partner
        d2 = d2 + dc * dc
    dist = jnp.sqrt(d2)
    v = d2 * (1.0 / (CUTOFF * CUTOFF))
    pw = 0.9999795145354539 + v * (-2.4663678376068665 + v * (
        2.0209826585045882 + v * (-0.6436782485800369 + v * 0.08910362041573856)))
    wc = jnp.where((d2 < CUTOFF * CUTOFF) & (eyef == 0.0),
                   pw, 0.0).astype(jnp.bfloat16)

    delta = CUTOFF / (G - 1)
    offs = lax.broadcasted_iota(jnp.int32, (1, G), 1).astype(jnp.float32) * delta
    coeff = -0.5 / (delta * delta)
    pieces = []
    for j in range(M):
        dj = dist[:, j:j + 1]
        arg = coeff * (dj - offs) ** 2
        pieces.append(jnp.exp(arg.astype(jnp.bfloat16)))
    rbf16 = jnp.concatenate(pieces, axis=0)

    tf = []
    for l in range(L):
        u = jnp.dot(rbf16, mw1_ref[l].astype(jnp.bfloat16),
                    preferred_element_type=jnp.float32)
        x = (u + mb1_ref[l:l + 1, :]).astype(jnp.bfloat16)
        s = (jnp.maximum(x, jnp.bfloat16(0.0)) - jnp.bfloat16(LN2)
             + jnp.log1p(jnp.exp(-jnp.abs(x))))
        t = jnp.dot(s, mw2_ref[l].astype(jnp.bfloat16),
                    preferred_element_type=jnp.float32)
        tf.append((t + mb2_ref[l:l + 1, :]).astype(jnp.bfloat16))

    S16 = S.astype(jnp.bfloat16)
    Tcat16 = Tcat.astype(jnp.bfloat16)
    Sw = [S16 * wc[:, j:j + 1] for j in range(M)]
    hh = h
    for l in range(L):
        xj = jnp.dot(hh.astype(jnp.bfloat16), cw1_ref[l].astype(jnp.bfloat16),
                     preferred_element_type=jnp.float32)
        xjsel16 = jnp.dot(Tcat16, xj.astype(jnp.bfloat16),
                          preferred_element_type=jnp.float32
                          ).astype(jnp.bfloat16)
        t = tf[l]
        msg16 = jnp.zeros((R, F), jnp.bfloat16)
        for j in range(M):
            xjw = jnp.dot(Sw[j], xjsel16[j * BM:(j + 1) * BM, :],
                          preferred_element_type=jnp.float32
                          ).astype(jnp.bfloat16)
            msg16 = msg16 + t[j * R:(j + 1) * R, :] * xjw
        x = (jnp.dot(msg16, cw2_ref[l].astype(jnp.bfloat16),
                     preferred_element_type=jnp.float32)
             + cb2_ref[l:l + 1, :]).astype(jnp.bfloat16)
        msg = (jnp.maximum(x, jnp.bfloat16(0.0)) - jnp.bfloat16(LN2)
               + jnp.log1p(jnp.exp(-jnp.abs(x))))
        msg = jnp.dot(msg, iw_ref[l].astype(jnp.bfloat16),
                      preferred_element_type=jnp.float32)
        hh = hh + msg + ib_ref[l:l + 1, :]

    o = _ssp(jnp.dot(hh, ow1_ref[...], preferred_element_type=jnp.float32)
             + ob1_ref[...])
    o = jnp.dot(o, ow2_ref[...], preferred_element_type=jnp.float32) + ob2_ref[...]
    out_ref[...] = jnp.dot(St, o, preferred_element_type=jnp.float32)


@jax.jit
def kernel(z, pos, batch, emb, mlp_w1, mlp_b1, mlp_w2, mlp_b2,
           conv_w1, conv_w2, conv_b2, int_w, int_b,
           out_w1, out_b1, out_w2, out_b2):
    del batch
    z2 = z.reshape(N, 1)
    posr = pos.T.reshape(3, N, 1)
    posm = pos.reshape(B, M, 3).transpose(2, 0, 1)
    ob1 = out_b1.reshape(1, H // 2)
    ob2 = out_b2.reshape(1, 1)

    r_i = jnp.arange(R, dtype=jnp.int32)
    mol = r_i // M
    atom = r_i % M
    S = (mol[:, None] == jnp.arange(BM, dtype=jnp.int32)[None, :]).astype(jnp.float32)
    q_j = jnp.arange(R, dtype=jnp.int32) // BM
    q_bm = jnp.arange(R, dtype=jnp.int32) % BM
    Tcat = ((mol[None, :] == q_bm[:, None])
            & (atom[None, :] == q_j[:, None])).astype(jnp.float32)
    St = S.T
    eyef = (atom[:, None] == jnp.arange(M, dtype=jnp.int32)[None, :]).astype(jnp.float32)

    full = lambda a: pl.BlockSpec(a.shape, lambda i: (0,) * a.ndim)
    out = pl.pallas_call(
        _block_kernel,
        grid=(GRID,),
        in_specs=[
            pl.BlockSpec((R, 1), lambda i: (i, 0)),
            pl.BlockSpec((3, R, 1), lambda i: (0, i, 0)),
            pl.BlockSpec((3, BM, M), lambda i: (0, i, 0)),
            full(S), full(Tcat), full(St), full(eyef),
            full(emb),
            full(mlp_w1), full(mlp_b1), full(mlp_w2), full(mlp_b2),
            full(conv_w1), full(conv_w2), full(conv_b2),
            full(int_w), full(int_b),
            full(out_w1), full(ob1), full(out_w2), full(ob2),
        ],
        out_specs=pl.BlockSpec((BM, 1), lambda i: (i, 0)),
        out_shape=jax.ShapeDtypeStruct((B, 1), jnp.float32),
        compiler_params=pltpu.CompilerParams(
            dimension_semantics=("parallel",)),
    )(z2, posr, posm, S, Tcat, St, eyef,
      emb, mlp_w1, mlp_b1, mlp_w2, mlp_b2,
      conv_w1, conv_w2, conv_b2, int_w, int_b, out_w1, ob1, out_w2, ob2)
    return out
